# baseline (speedup 1.0000x reference)
import jax
import jax.numpy as jnp
from jax import lax
from jax.experimental import pallas as pl
from jax.experimental.pallas import tpu as pltpu
from jax.experimental.pallas import tpu_sc as plsc

B = 4096
L = 50
D = 32
NC = 2
NS = 16
NW = NC * NS
BPW = B // NW
CHUNK = 32
NCHUNK = BPW // CHUNK
F32_MAX = 3.4028235e38


def _line16(v):
    lvb = VB.bit_length() - 1
    lsub = SUB.bit_length() - 1
    return (((v >> lvb) << (lsub + 3)) + ((v & (SUB - 1)) << 3)
            + (((v >> lsub) & 3) << 1))


def _sc_body(text_ref, ai0_ref, ai1_ref, ai2_ref, emb_ref, cat0_ref, cat1_ref,
             cat2_ref, sum_ref, cat_ref, idx_v, lidxa_v, lidxb_v, rowsa_v,
             rowsb_v, cat_idx_v, clidxa_v, clidxb_v, cat_rowsa_v, cat_rowsb_v,
             sum_v, catsum_v, gsem, csem):
    wid = lax.axis_index("s") * NC + lax.axis_index("c")
    base = wid * BPW

    pltpu.sync_copy(text_ref.at[pl.ds(base, BPW)], idx_v)
    ai_refs = (ai0_ref, ai1_ref, ai2_ref)
    for c in range(3):
        pltpu.sync_copy(ai_refs[c].at[pl.ds(base, BPW)], cat_idx_v.at[c])

    for c in range(3):
        for g in range(0, BPW, 16):
            la = _line16(cat_idx_v[c, pl.ds(g, 16)])
            clidxa_v[c, pl.ds(g, 16)] = la
            clidxb_v[c, pl.ds(g, 16)] = la + 1

    cat_tables = (cat0_ref, cat1_ref, cat2_ref)
    cat_descs = [
        pltpu.async_copy(cat_tables[c].at[clidxa_v.at[c]], cat_rowsa_v.at[c], csem)
        for c in range(3)
    ] + [
        pltpu.async_copy(cat_tables[c].at[clidxb_v.at[c]], cat_rowsb_v.at[c], csem)
        for c in range(3)
    ]

    def lidx_body(r, carry):
        for g in (0, 16, 32, L - 16):
            la = _line16(idx_v[r, pl.ds(g, 16)])
            lidxa_v[r, pl.ds(g, 16)] = la
            lidxb_v[r, pl.ds(g, 16)] = la + 1
        return carry

    lax.fori_loop(0, BPW, lidx_body, jnp.int32(0))

    def compute_row(r, chunk_base):
        a0 = [jnp.zeros((16,), jnp.float32) for _ in range(4)]
        a1 = [jnp.zeros((16,), jnp.float32) for _ in range(4)]
        for t in range(L):
            a0[t % 4] = a0[t % 4] + rowsa_v[r, t, :]
            a1[t % 4] = a1[t % 4] + rowsb_v[r, t, :]
        row = chunk_base + r
        sum_v[row, pl.ds(0, 16)] = (a0[0] + a0[1]) + (a0[2] + a0[3])
        sum_v[row, pl.ds(16, 16)] = (a1[0] + a1[1]) + (a1[2] + a1[3])

    for chunk in range(NCHUNK):
        cb = chunk * CHUNK
        descs = [
            pltpu.async_copy(emb_ref.at[lidxa_v.at[cb + r]], rowsa_v.at[r], gsem)
            for r in range(CHUNK)
        ] + [
            pltpu.async_copy(emb_ref.at[lidxb_v.at[cb + r]], rowsb_v.at[r], gsem)
            for r in range(CHUNK)
        ]
        for d in descs:
            d.wait()

        def body(r, carry):
            compute_row(r, cb)
            return carry

        lax.fori_loop(0, CHUNK, body, jnp.int32(0))

    for d in cat_descs:
        d.wait()

    def cat_body(r, carry):
        catsum_v[r, pl.ds(0, 16)] = (
            cat_rowsa_v[0, r, :] + cat_rowsa_v[1, r, :] + cat_rowsa_v[2, r, :]
        )
        catsum_v[r, pl.ds(16, 16)] = (
            cat_rowsb_v[0, r, :] + cat_rowsb_v[1, r, :] + cat_rowsb_v[2, r, :]
        )
        return carry

    lax.fori_loop(0, BPW, cat_body, jnp.int32(0))

    pltpu.sync_copy(sum_v, sum_ref.at[pl.ds(base, BPW)])
    pltpu.sync_copy(catsum_v, cat_ref.at[pl.ds(base, BPW)])


@jax.jit
def _sc_pool(encoded_text, ai0, ai1, ai2, emb_lines, cat_emb0, cat_emb1,
             cat_emb2):
    mesh = plsc.VectorSubcoreMesh(
        core_axis_name="c", subcore_axis_name="s", num_cores=NC, num_subcores=NS
    )
    return pl.kernel(
        _sc_body,
        out_type=(
            jax.ShapeDtypeStruct((B, D), jnp.float32),
            jax.ShapeDtypeStruct((B, D), jnp.float32),
        ),
        mesh=mesh,
        compiler_params=pltpu.CompilerParams(use_tc_tiling_on_sc=False),
        scratch_types=[
            pltpu.VMEM((BPW, L), jnp.int32),
            pltpu.VMEM((BPW, L), jnp.int32),
            pltpu.VMEM((BPW, L), jnp.int32),
            pltpu.VMEM((CHUNK, L, 16), jnp.float32),
            pltpu.VMEM((CHUNK, L, 16), jnp.float32),
            pltpu.VMEM((3, BPW), jnp.int32),
            pltpu.VMEM((3, BPW), jnp.int32),
            pltpu.VMEM((3, BPW), jnp.int32),
            pltpu.VMEM((3, BPW, 16), jnp.float32),
            pltpu.VMEM((3, BPW, 16), jnp.float32),
            pltpu.VMEM((BPW, D), jnp.float32),
            pltpu.VMEM((BPW, D), jnp.float32),
            pltpu.SemaphoreType.DMA,
            pltpu.SemaphoreType.DMA,
        ],
    )(encoded_text, ai0, ai1, ai2, emb_lines, cat_emb0, cat_emb1, cat_emb2)


VB = 4096
SUB = VB // 4


def _tr_body(x_ref, o_ref):
    x = x_ref[...]
    x4 = jnp.concatenate([x[:, j * SUB:(j + 1) * SUB] for j in range(4)],
                         axis=0)
    eye = (lax.broadcasted_iota(jnp.int32, (128, 128), 0)
           == lax.broadcasted_iota(jnp.int32, (128, 128), 1)).astype(jnp.float32)
    o_ref[...] = lax.dot_general(
        x4, eye, (((0,), (0,)), ((), ())),
        preferred_element_type=jnp.float32,
    )


@jax.jit
def _tc_transpose_pack(table_t):
    d, v = table_t.shape
    grid = (v + VB - 1) // VB
    return pl.pallas_call(
        _tr_body,
        grid=(grid,),
        in_specs=[pl.BlockSpec((d, VB), lambda i: (0, i))],
        out_specs=pl.BlockSpec((SUB, 128), lambda i: (i, 0)),
        out_shape=jax.ShapeDtypeStruct((grid * SUB, 128), jnp.float32),
    )(table_t)


def _head_body(text_ref, sum_ref, cat_ref, w_ref, b_ref, o_ref):
    cnt = jnp.sum((text_ref[...] != 0).astype(jnp.float32), axis=1,
                  keepdims=True)
    x = sum_ref[...] / cnt
    x = jnp.where(x != x, jnp.float32(0.0), x)
    x = jnp.minimum(jnp.maximum(x, -F32_MAX), F32_MAX)
    x = x + cat_ref[...]
    o_ref[...] = (
        jnp.dot(x, w_ref[...], preferred_element_type=jnp.float32) + b_ref[...]
    )


@jax.jit
def _tc_head(encoded_text, x_sum, cat_sum, w, b2d):
    bm = 512
    nc = w.shape[1]
    return pl.pallas_call(
        _head_body,
        grid=(B // bm,),
        in_specs=[
            pl.BlockSpec((bm, L), lambda i: (i, 0)),
            pl.BlockSpec((bm, D), lambda i: (i, 0)),
            pl.BlockSpec((bm, D), lambda i: (i, 0)),
            pl.BlockSpec((D, nc), lambda i: (0, 0)),
            pl.BlockSpec((1, nc), lambda i: (0, 0)),
        ],
        out_specs=pl.BlockSpec((bm, nc), lambda i: (i, 0)),
        out_shape=jax.ShapeDtypeStruct((B, nc), jnp.float32),
    )(encoded_text, x_sum, cat_sum, w, b2d)


def kernel(encoded_text, additional_inputs, emb_table, cat_emb0, cat_emb1,
           cat_emb2, fc_w, fc_b):
    text = encoded_text.astype(jnp.int32)
    ai = additional_inputs.astype(jnp.int32)
    emb16 = _tc_transpose_pack(emb_table.T).reshape(-1, 16)
    cat16 = [_tc_transpose_pack(t.T).reshape(-1, 16)
             for t in (cat_emb0, cat_emb1, cat_emb2)]
    x_sum, cat_sum = _sc_pool(text, ai[:, 0], ai[:, 1], ai[:, 2], emb16,
                              cat16[0], cat16[1], cat16[2])
    return _tc_head(text, x_sum, cat_sum, fc_w, fc_b.reshape(1, -1))


# speedup vs baseline: 1.3078x; 1.3078x over previous
import jax
import jax.numpy as jnp
from jax import lax
from jax.experimental import pallas as pl
from jax.experimental.pallas import tpu as pltpu
from jax.experimental.pallas import tpu_sc as plsc

B = 4096
L = 50
D = 32
NC = 2
NS = 16
NW = NC * NS
BPW = B // NW
CHUNK = 32
NCHUNK = BPW // CHUNK
F32_MAX = 3.4028235e38


def _line16(v):
    lvb = VB.bit_length() - 1
    lsub = SUB.bit_length() - 1
    return (((v >> lvb) << (lsub + 3)) + ((v & (SUB - 1)) << 3)
            + (((v >> lsub) & 3) << 1))


def _sc_body(text_ref, ai0_ref, ai1_ref, ai2_ref, emb_ref, cat0_ref, cat1_ref,
             cat2_ref, sum_ref, cat_ref, idx_v, lidxa_v, lidxb_v, rowsa_v,
             rowsb_v, cat_idx_v, clidxa_v, clidxb_v, cat_rowsa_v, cat_rowsb_v,
             sum_v, catsum_v, gsem, csem):
    wid = lax.axis_index("s") * NC + lax.axis_index("c")
    base = wid * BPW

    pltpu.sync_copy(text_ref.at[pl.ds(base, BPW)], idx_v)
    ai_refs = (ai0_ref, ai1_ref, ai2_ref)
    for c in range(3):
        pltpu.sync_copy(ai_refs[c].at[pl.ds(base, BPW)], cat_idx_v.at[c])

    for c in range(3):
        for g in range(0, BPW, 16):
            la = _line16(cat_idx_v[c, pl.ds(g, 16)])
            clidxa_v[c, pl.ds(g, 16)] = la
            clidxb_v[c, pl.ds(g, 16)] = la + 1

    cat_tables = (cat0_ref, cat1_ref, cat2_ref)
    cat_descs = [
        pltpu.async_copy(cat_tables[c].at[clidxa_v.at[c]], cat_rowsa_v.at[c], csem)
        for c in range(3)
    ] + [
        pltpu.async_copy(cat_tables[c].at[clidxb_v.at[c]], cat_rowsb_v.at[c], csem)
        for c in range(3)
    ]

    def lidx_body(r, carry):
        for g in (0, 16, 32, L - 16):
            la = _line16(idx_v[r, pl.ds(g, 16)])
            lidxa_v[r, pl.ds(g, 16)] = la
            lidxb_v[r, pl.ds(g, 16)] = la + 1
        return carry

    lax.fori_loop(0, BPW, lidx_body, jnp.int32(0))

    def compute_row(r, chunk_base):
        a0 = [jnp.zeros((16,), jnp.float32) for _ in range(4)]
        a1 = [jnp.zeros((16,), jnp.float32) for _ in range(4)]
        for t in range(L):
            a0[t % 4] = a0[t % 4] + rowsa_v[r, t, :]
            a1[t % 4] = a1[t % 4] + rowsb_v[r, t, :]
        row = chunk_base + r
        sum_v[row, pl.ds(0, 16)] = (a0[0] + a0[1]) + (a0[2] + a0[3])
        sum_v[row, pl.ds(16, 16)] = (a1[0] + a1[1]) + (a1[2] + a1[3])

    for chunk in range(NCHUNK):
        cb = chunk * CHUNK
        descs = [
            pltpu.async_copy(emb_ref.at[lidxa_v.at[cb + r]], rowsa_v.at[r], gsem)
            for r in range(CHUNK)
        ] + [
            pltpu.async_copy(emb_ref.at[lidxb_v.at[cb + r]], rowsb_v.at[r], gsem)
            for r in range(CHUNK)
        ]
        for d in descs:
            d.wait()

        def body(r, carry):
            compute_row(r, cb)
            return carry

        lax.fori_loop(0, CHUNK, body, jnp.int32(0))

    for d in cat_descs:
        d.wait()

    def cat_body(r, carry):
        catsum_v[r, pl.ds(0, 16)] = (
            cat_rowsa_v[0, r, :] + cat_rowsa_v[1, r, :] + cat_rowsa_v[2, r, :]
        )
        catsum_v[r, pl.ds(16, 16)] = (
            cat_rowsb_v[0, r, :] + cat_rowsb_v[1, r, :] + cat_rowsb_v[2, r, :]
        )
        return carry

    lax.fori_loop(0, BPW, cat_body, jnp.int32(0))

    pltpu.sync_copy(sum_v, sum_ref.at[pl.ds(base, BPW)])
    pltpu.sync_copy(catsum_v, cat_ref.at[pl.ds(base, BPW)])


@jax.jit
def _sc_pool(encoded_text, ai0, ai1, ai2, emb_lines, cat_emb0, cat_emb1,
             cat_emb2):
    mesh = plsc.VectorSubcoreMesh(
        core_axis_name="c", subcore_axis_name="s", num_cores=NC, num_subcores=NS
    )
    return pl.kernel(
        _sc_body,
        out_type=(
            jax.ShapeDtypeStruct((B, D), jnp.float32),
            jax.ShapeDtypeStruct((B, D), jnp.float32),
        ),
        mesh=mesh,
        compiler_params=pltpu.CompilerParams(use_tc_tiling_on_sc=False),
        scratch_types=[
            pltpu.VMEM((BPW, L), jnp.int32),
            pltpu.VMEM((BPW, L), jnp.int32),
            pltpu.VMEM((BPW, L), jnp.int32),
            pltpu.VMEM((CHUNK, L, 16), jnp.float32),
            pltpu.VMEM((CHUNK, L, 16), jnp.float32),
            pltpu.VMEM((3, BPW), jnp.int32),
            pltpu.VMEM((3, BPW), jnp.int32),
            pltpu.VMEM((3, BPW), jnp.int32),
            pltpu.VMEM((3, BPW, 16), jnp.float32),
            pltpu.VMEM((3, BPW, 16), jnp.float32),
            pltpu.VMEM((BPW, D), jnp.float32),
            pltpu.VMEM((BPW, D), jnp.float32),
            pltpu.SemaphoreType.DMA,
            pltpu.SemaphoreType.DMA,
        ],
    )(encoded_text, ai0, ai1, ai2, emb_lines, cat_emb0, cat_emb1, cat_emb2)


VB = 8192
SUB = VB // 4


def _tr_body(x_ref, o_ref):
    x = x_ref[...]
    x4 = jnp.concatenate([x[:, j * SUB:(j + 1) * SUB] for j in range(4)],
                         axis=0)
    eye = (lax.broadcasted_iota(jnp.int32, (128, 128), 0)
           == lax.broadcasted_iota(jnp.int32, (128, 128), 1)).astype(jnp.float32)
    o_ref[...] = lax.dot_general(
        x4, eye, (((0,), (0,)), ((), ())),
        preferred_element_type=jnp.float32,
    )


@jax.jit
def _tc_transpose_pack(table_t):
    d, v = table_t.shape
    grid = (v + VB - 1) // VB
    return pl.pallas_call(
        _tr_body,
        grid=(grid,),
        in_specs=[pl.BlockSpec((d, VB), lambda i: (0, i))],
        out_specs=pl.BlockSpec((SUB, 128), lambda i: (i, 0)),
        out_shape=jax.ShapeDtypeStruct((grid * SUB, 128), jnp.float32),
    )(table_t)


def _head_body(text_ref, sum_ref, cat_ref, w_ref, b_ref, o_ref):
    cnt = jnp.sum((text_ref[...] != 0).astype(jnp.float32), axis=1,
                  keepdims=True)
    x = sum_ref[...] / cnt
    x = jnp.where(x != x, jnp.float32(0.0), x)
    x = jnp.minimum(jnp.maximum(x, -F32_MAX), F32_MAX)
    x = x + cat_ref[...]
    o_ref[...] = (
        jnp.dot(x, w_ref[...], preferred_element_type=jnp.float32) + b_ref[...]
    )


@jax.jit
def _tc_head(encoded_text, x_sum, cat_sum, w, b2d):
    bm = 512
    nc = w.shape[1]
    return pl.pallas_call(
        _head_body,
        grid=(B // bm,),
        in_specs=[
            pl.BlockSpec((bm, L), lambda i: (i, 0)),
            pl.BlockSpec((bm, D), lambda i: (i, 0)),
            pl.BlockSpec((bm, D), lambda i: (i, 0)),
            pl.BlockSpec((D, nc), lambda i: (0, 0)),
            pl.BlockSpec((1, nc), lambda i: (0, 0)),
        ],
        out_specs=pl.BlockSpec((bm, nc), lambda i: (i, 0)),
        out_shape=jax.ShapeDtypeStruct((B, nc), jnp.float32),
    )(encoded_text, x_sum, cat_sum, w, b2d)


def kernel(encoded_text, additional_inputs, emb_table, cat_emb0, cat_emb1,
           cat_emb2, fc_w, fc_b):
    text = encoded_text.astype(jnp.int32)
    ai = additional_inputs.astype(jnp.int32)
    emb16 = _tc_transpose_pack(emb_table.T).reshape(-1, 16)
    cat16 = [_tc_transpose_pack(t.T).reshape(-1, 16)
             for t in (cat_emb0, cat_emb1, cat_emb2)]
    x_sum, cat_sum = _sc_pool(text, ai[:, 0], ai[:, 1], ai[:, 2], emb16,
                              cat16[0], cat16[1], cat16[2])
    return _tc_head(text, x_sum, cat_sum, fc_w, fc_b.reshape(1, -1))


# speedup vs baseline: 1.5836x; 1.2109x over previous
import jax
import jax.numpy as jnp
from jax import lax
from jax.experimental import pallas as pl
from jax.experimental.pallas import tpu as pltpu
from jax.experimental.pallas import tpu_sc as plsc

B = 4096
L = 50
D = 32
NC = 2
NS = 16
NW = NC * NS
BPW = B // NW
CHUNK = 32
NCHUNK = BPW // CHUNK
F32_MAX = 3.4028235e38


def _line16(v):
    lvb = VB.bit_length() - 1
    lsub = SUB.bit_length() - 1
    return (((v >> lvb) << (lsub + 3)) + ((v & (SUB - 1)) << 3)
            + (((v >> lsub) & 3) << 1))


def _sc_body(text_ref, ai0_ref, ai1_ref, ai2_ref, emb_ref, cat0_ref, cat1_ref,
             cat2_ref, sum_ref, cat_ref, idx_v, lidxa_v, lidxb_v, rowsa_v,
             rowsb_v, cat_idx_v, clidxa_v, clidxb_v, cat_rowsa_v, cat_rowsb_v,
             sum_v, catsum_v, gsem, csem):
    wid = lax.axis_index("s") * NC + lax.axis_index("c")
    base = wid * BPW

    pltpu.sync_copy(text_ref.at[pl.ds(base, BPW)], idx_v)
    ai_refs = (ai0_ref, ai1_ref, ai2_ref)
    for c in range(3):
        pltpu.sync_copy(ai_refs[c].at[pl.ds(base, BPW)], cat_idx_v.at[c])

    for c in range(3):
        for g in range(0, BPW, 16):
            la = _line16(cat_idx_v[c, pl.ds(g, 16)])
            clidxa_v[c, pl.ds(g, 16)] = la
            clidxb_v[c, pl.ds(g, 16)] = la + 1

    cat_tables = (cat0_ref, cat1_ref, cat2_ref)
    cat_descs = [
        pltpu.async_copy(cat_tables[c].at[clidxa_v.at[c]], cat_rowsa_v.at[c], csem)
        for c in range(3)
    ] + [
        pltpu.async_copy(cat_tables[c].at[clidxb_v.at[c]], cat_rowsb_v.at[c], csem)
        for c in range(3)
    ]

    def lidx_body(r, carry):
        for g in (0, 16, 32, L - 16):
            la = _line16(idx_v[r, pl.ds(g, 16)])
            lidxa_v[r, pl.ds(g, 16)] = la
            lidxb_v[r, pl.ds(g, 16)] = la + 1
        return carry

    lax.fori_loop(0, BPW, lidx_body, jnp.int32(0))

    def compute_row(r, chunk_base):
        a0 = [jnp.zeros((16,), jnp.float32) for _ in range(4)]
        a1 = [jnp.zeros((16,), jnp.float32) for _ in range(4)]
        for t in range(L):
            a0[t % 4] = a0[t % 4] + rowsa_v[r, t, :]
            a1[t % 4] = a1[t % 4] + rowsb_v[r, t, :]
        row = chunk_base + r
        sum_v[row, pl.ds(0, 16)] = (a0[0] + a0[1]) + (a0[2] + a0[3])
        sum_v[row, pl.ds(16, 16)] = (a1[0] + a1[1]) + (a1[2] + a1[3])

    for chunk in range(NCHUNK):
        cb = chunk * CHUNK
        descs = [
            pltpu.async_copy(emb_ref.at[lidxa_v.at[cb + r]], rowsa_v.at[r], gsem)
            for r in range(CHUNK)
        ] + [
            pltpu.async_copy(emb_ref.at[lidxb_v.at[cb + r]], rowsb_v.at[r], gsem)
            for r in range(CHUNK)
        ]
        for d in descs:
            d.wait()

        def body(r, carry):
            compute_row(r, cb)
            return carry

        lax.fori_loop(0, CHUNK, body, jnp.int32(0))

    for d in cat_descs:
        d.wait()

    def cat_body(r, carry):
        catsum_v[r, pl.ds(0, 16)] = (
            cat_rowsa_v[0, r, :] + cat_rowsa_v[1, r, :] + cat_rowsa_v[2, r, :]
        )
        catsum_v[r, pl.ds(16, 16)] = (
            cat_rowsb_v[0, r, :] + cat_rowsb_v[1, r, :] + cat_rowsb_v[2, r, :]
        )
        return carry

    lax.fori_loop(0, BPW, cat_body, jnp.int32(0))

    pltpu.sync_copy(sum_v, sum_ref.at[pl.ds(base, BPW)])
    pltpu.sync_copy(catsum_v, cat_ref.at[pl.ds(base, BPW)])


@jax.jit
def _sc_pool(encoded_text, ai0, ai1, ai2, emb_lines, cat_emb0, cat_emb1,
             cat_emb2):
    mesh = plsc.VectorSubcoreMesh(
        core_axis_name="c", subcore_axis_name="s", num_cores=NC, num_subcores=NS
    )
    return pl.kernel(
        _sc_body,
        out_type=(
            jax.ShapeDtypeStruct((B, D), jnp.float32),
            jax.ShapeDtypeStruct((B, D), jnp.float32),
        ),
        mesh=mesh,
        compiler_params=pltpu.CompilerParams(use_tc_tiling_on_sc=False),
        scratch_types=[
            pltpu.VMEM((BPW, L), jnp.int32),
            pltpu.VMEM((BPW, L), jnp.int32),
            pltpu.VMEM((BPW, L), jnp.int32),
            pltpu.VMEM((CHUNK, L, 16), jnp.float32),
            pltpu.VMEM((CHUNK, L, 16), jnp.float32),
            pltpu.VMEM((3, BPW), jnp.int32),
            pltpu.VMEM((3, BPW), jnp.int32),
            pltpu.VMEM((3, BPW), jnp.int32),
            pltpu.VMEM((3, BPW, 16), jnp.float32),
            pltpu.VMEM((3, BPW, 16), jnp.float32),
            pltpu.VMEM((BPW, D), jnp.float32),
            pltpu.VMEM((BPW, D), jnp.float32),
            pltpu.SemaphoreType.DMA,
            pltpu.SemaphoreType.DMA,
        ],
    )(encoded_text, ai0, ai1, ai2, emb_lines, cat_emb0, cat_emb1, cat_emb2)


VB = 16384
SUB = VB // 4


def _tr_body(x_ref, o_ref):
    x = x_ref[...]
    x4 = jnp.concatenate([x[:, j * SUB:(j + 1) * SUB] for j in range(4)],
                         axis=0)
    eye = (lax.broadcasted_iota(jnp.int32, (128, 128), 0)
           == lax.broadcasted_iota(jnp.int32, (128, 128), 1)).astype(jnp.float32)
    o_ref[...] = lax.dot_general(
        x4, eye, (((0,), (0,)), ((), ())),
        preferred_element_type=jnp.float32,
    )


@jax.jit
def _tc_transpose_pack(table_t):
    d, v = table_t.shape
    grid = (v + VB - 1) // VB
    return pl.pallas_call(
        _tr_body,
        grid=(grid,),
        in_specs=[pl.BlockSpec((d, VB), lambda i: (0, i))],
        out_specs=pl.BlockSpec((SUB, 128), lambda i: (i, 0)),
        out_shape=jax.ShapeDtypeStruct((grid * SUB, 128), jnp.float32),
    )(table_t)


def _head_body(text_ref, sum_ref, cat_ref, w_ref, b_ref, o_ref):
    cnt = jnp.sum((text_ref[...] != 0).astype(jnp.float32), axis=1,
                  keepdims=True)
    x = sum_ref[...] / cnt
    x = jnp.where(x != x, jnp.float32(0.0), x)
    x = jnp.minimum(jnp.maximum(x, -F32_MAX), F32_MAX)
    x = x + cat_ref[...]
    o_ref[...] = (
        jnp.dot(x, w_ref[...], preferred_element_type=jnp.float32) + b_ref[...]
    )


@jax.jit
def _tc_head(encoded_text, x_sum, cat_sum, w, b2d):
    bm = 512
    nc = w.shape[1]
    return pl.pallas_call(
        _head_body,
        grid=(B // bm,),
        in_specs=[
            pl.BlockSpec((bm, L), lambda i: (i, 0)),
            pl.BlockSpec((bm, D), lambda i: (i, 0)),
            pl.BlockSpec((bm, D), lambda i: (i, 0)),
            pl.BlockSpec((D, nc), lambda i: (0, 0)),
            pl.BlockSpec((1, nc), lambda i: (0, 0)),
        ],
        out_specs=pl.BlockSpec((bm, nc), lambda i: (i, 0)),
        out_shape=jax.ShapeDtypeStruct((B, nc), jnp.float32),
    )(encoded_text, x_sum, cat_sum, w, b2d)


def kernel(encoded_text, additional_inputs, emb_table, cat_emb0, cat_emb1,
           cat_emb2, fc_w, fc_b):
    text = encoded_text.astype(jnp.int32)
    ai = additional_inputs.astype(jnp.int32)
    emb16 = _tc_transpose_pack(emb_table.T).reshape(-1, 16)
    cat16 = [_tc_transpose_pack(t.T).reshape(-1, 16)
             for t in (cat_emb0, cat_emb1, cat_emb2)]
    x_sum, cat_sum = _sc_pool(text, ai[:, 0], ai[:, 1], ai[:, 2], emb16,
                              cat16[0], cat16[1], cat16[2])
    return _tc_head(text, x_sum, cat_sum, fc_w, fc_b.reshape(1, -1))


# speedup vs baseline: 1.7017x; 1.0746x over previous
import jax
import jax.numpy as jnp
from jax import lax
from jax.experimental import pallas as pl
from jax.experimental.pallas import tpu as pltpu
from jax.experimental.pallas import tpu_sc as plsc

B = 4096
L = 50
D = 32
NC = 2
NS = 16
NW = NC * NS
BPW = B // NW
CHUNK = 32
NCHUNK = BPW // CHUNK
F32_MAX = 3.4028235e38


def _line16(v):
    lvb = VB.bit_length() - 1
    lsub = SUB.bit_length() - 1
    return (((v >> lvb) << (lsub + 3)) + ((v & (SUB - 1)) << 3)
            + (((v >> lsub) & 3) << 1))


def _sc_body(text_ref, ai0_ref, ai1_ref, ai2_ref, emb_ref, cat0_ref, cat1_ref,
             cat2_ref, sum_ref, cat_ref, idx_v, lidxa_v, lidxb_v, rowsa_v,
             rowsb_v, cat_idx_v, clidxa_v, clidxb_v, cat_rowsa_v, cat_rowsb_v,
             sum_v, catsum_v, gsem, csem):
    wid = lax.axis_index("s") * NC + lax.axis_index("c")
    base = wid * BPW

    pltpu.sync_copy(text_ref.at[pl.ds(base, BPW)], idx_v)
    ai_refs = (ai0_ref, ai1_ref, ai2_ref)
    for c in range(3):
        pltpu.sync_copy(ai_refs[c].at[pl.ds(base, BPW)], cat_idx_v.at[c])

    for c in range(3):
        for g in range(0, BPW, 16):
            la = _line16(cat_idx_v[c, pl.ds(g, 16)])
            clidxa_v[c, pl.ds(g, 16)] = la
            clidxb_v[c, pl.ds(g, 16)] = la + 1

    cat_tables = (cat0_ref, cat1_ref, cat2_ref)
    cat_descs = [
        pltpu.async_copy(cat_tables[c].at[clidxa_v.at[c]], cat_rowsa_v.at[c], csem)
        for c in range(3)
    ] + [
        pltpu.async_copy(cat_tables[c].at[clidxb_v.at[c]], cat_rowsb_v.at[c], csem)
        for c in range(3)
    ]

    def lidx_body(r, carry):
        for g in (0, 16, 32, L - 16):
            la = _line16(idx_v[r, pl.ds(g, 16)])
            lidxa_v[r, pl.ds(g, 16)] = la
            lidxb_v[r, pl.ds(g, 16)] = la + 1
        return carry

    lax.fori_loop(0, BPW, lidx_body, jnp.int32(0))

    def compute_row(r, chunk_base):
        a0 = [jnp.zeros((16,), jnp.float32) for _ in range(4)]
        a1 = [jnp.zeros((16,), jnp.float32) for _ in range(4)]
        for t in range(L):
            a0[t % 4] = a0[t % 4] + rowsa_v[r, t, :]
            a1[t % 4] = a1[t % 4] + rowsb_v[r, t, :]
        row = chunk_base + r
        sum_v[row, pl.ds(0, 16)] = (a0[0] + a0[1]) + (a0[2] + a0[3])
        sum_v[row, pl.ds(16, 16)] = (a1[0] + a1[1]) + (a1[2] + a1[3])

    for chunk in range(NCHUNK):
        cb = chunk * CHUNK
        descs = [
            pltpu.async_copy(emb_ref.at[lidxa_v.at[cb + r]], rowsa_v.at[r], gsem)
            for r in range(CHUNK)
        ] + [
            pltpu.async_copy(emb_ref.at[lidxb_v.at[cb + r]], rowsb_v.at[r], gsem)
            for r in range(CHUNK)
        ]
        for d in descs:
            d.wait()

        def body(r, carry):
            compute_row(r, cb)
            return carry

        lax.fori_loop(0, CHUNK, body, jnp.int32(0))

    for d in cat_descs:
        d.wait()

    def cat_body(r, carry):
        catsum_v[r, pl.ds(0, 16)] = (
            cat_rowsa_v[0, r, :] + cat_rowsa_v[1, r, :] + cat_rowsa_v[2, r, :]
        )
        catsum_v[r, pl.ds(16, 16)] = (
            cat_rowsb_v[0, r, :] + cat_rowsb_v[1, r, :] + cat_rowsb_v[2, r, :]
        )
        return carry

    lax.fori_loop(0, BPW, cat_body, jnp.int32(0))

    pltpu.sync_copy(sum_v, sum_ref.at[pl.ds(base, BPW)])
    pltpu.sync_copy(catsum_v, cat_ref.at[pl.ds(base, BPW)])


@jax.jit
def _sc_pool(encoded_text, ai0, ai1, ai2, emb_lines, cat_emb0, cat_emb1,
             cat_emb2):
    mesh = plsc.VectorSubcoreMesh(
        core_axis_name="c", subcore_axis_name="s", num_cores=NC, num_subcores=NS
    )
    return pl.kernel(
        _sc_body,
        out_type=(
            jax.ShapeDtypeStruct((B, D), jnp.float32),
            jax.ShapeDtypeStruct((B, D), jnp.float32),
        ),
        mesh=mesh,
        compiler_params=pltpu.CompilerParams(use_tc_tiling_on_sc=False),
        scratch_types=[
            pltpu.VMEM((BPW, L), jnp.int32),
            pltpu.VMEM((BPW, L), jnp.int32),
            pltpu.VMEM((BPW, L), jnp.int32),
            pltpu.VMEM((CHUNK, L, 16), jnp.float32),
            pltpu.VMEM((CHUNK, L, 16), jnp.float32),
            pltpu.VMEM((3, BPW), jnp.int32),
            pltpu.VMEM((3, BPW), jnp.int32),
            pltpu.VMEM((3, BPW), jnp.int32),
            pltpu.VMEM((3, BPW, 16), jnp.float32),
            pltpu.VMEM((3, BPW, 16), jnp.float32),
            pltpu.VMEM((BPW, D), jnp.float32),
            pltpu.VMEM((BPW, D), jnp.float32),
            pltpu.SemaphoreType.DMA,
            pltpu.SemaphoreType.DMA,
        ],
    )(encoded_text, ai0, ai1, ai2, emb_lines, cat_emb0, cat_emb1, cat_emb2)


VB = 32768
SUB = VB // 4


def _tr_body(x_ref, o_ref):
    x = x_ref[...]
    x4 = jnp.concatenate([x[:, j * SUB:(j + 1) * SUB] for j in range(4)],
                         axis=0)
    eye = (lax.broadcasted_iota(jnp.int32, (128, 128), 0)
           == lax.broadcasted_iota(jnp.int32, (128, 128), 1)).astype(jnp.float32)
    o_ref[...] = lax.dot_general(
        x4, eye, (((0,), (0,)), ((), ())),
        preferred_element_type=jnp.float32,
    )


@jax.jit
def _tc_transpose_pack(table_t):
    d, v = table_t.shape
    grid = (v + VB - 1) // VB
    return pl.pallas_call(
        _tr_body,
        grid=(grid,),
        in_specs=[pl.BlockSpec((d, VB), lambda i: (0, i))],
        out_specs=pl.BlockSpec((SUB, 128), lambda i: (i, 0)),
        out_shape=jax.ShapeDtypeStruct((grid * SUB, 128), jnp.float32),
    )(table_t)


def _head_body(text_ref, sum_ref, cat_ref, w_ref, b_ref, o_ref):
    cnt = jnp.sum((text_ref[...] != 0).astype(jnp.float32), axis=1,
                  keepdims=True)
    x = sum_ref[...] / cnt
    x = jnp.where(x != x, jnp.float32(0.0), x)
    x = jnp.minimum(jnp.maximum(x, -F32_MAX), F32_MAX)
    x = x + cat_ref[...]
    o_ref[...] = (
        jnp.dot(x, w_ref[...], preferred_element_type=jnp.float32) + b_ref[...]
    )


@jax.jit
def _tc_head(encoded_text, x_sum, cat_sum, w, b2d):
    bm = 512
    nc = w.shape[1]
    return pl.pallas_call(
        _head_body,
        grid=(B // bm,),
        in_specs=[
            pl.BlockSpec((bm, L), lambda i: (i, 0)),
            pl.BlockSpec((bm, D), lambda i: (i, 0)),
            pl.BlockSpec((bm, D), lambda i: (i, 0)),
            pl.BlockSpec((D, nc), lambda i: (0, 0)),
            pl.BlockSpec((1, nc), lambda i: (0, 0)),
        ],
        out_specs=pl.BlockSpec((bm, nc), lambda i: (i, 0)),
        out_shape=jax.ShapeDtypeStruct((B, nc), jnp.float32),
    )(encoded_text, x_sum, cat_sum, w, b2d)


def kernel(encoded_text, additional_inputs, emb_table, cat_emb0, cat_emb1,
           cat_emb2, fc_w, fc_b):
    text = encoded_text.astype(jnp.int32)
    ai = additional_inputs.astype(jnp.int32)
    emb16 = _tc_transpose_pack(emb_table.T).reshape(-1, 16)
    cat16 = [_tc_transpose_pack(t.T).reshape(-1, 16)
             for t in (cat_emb0, cat_emb1, cat_emb2)]
    x_sum, cat_sum = _sc_pool(text, ai[:, 0], ai[:, 1], ai[:, 2], emb16,
                              cat16[0], cat16[1], cat16[2])
    return _tc_head(text, x_sum, cat_sum, fc_w, fc_b.reshape(1, -1))


# speedup vs baseline: 1.7202x; 1.0108x over previous
import jax
import jax.numpy as jnp
from jax import lax
from jax.experimental import pallas as pl
from jax.experimental.pallas import tpu as pltpu
from jax.experimental.pallas import tpu_sc as plsc

B = 4096
L = 50
D = 32
NC = 2
NS = 16
NW = NC * NS
BPW = B // NW
CHUNK = 32
NCHUNK = BPW // CHUNK
F32_MAX = 3.4028235e38


def _line16(v):
    lvb = VB.bit_length() - 1
    lsub = SUB.bit_length() - 1
    return (((v >> lvb) << (lsub + 3)) + ((v & (SUB - 1)) << 3)
            + (((v >> lsub) & 3) << 1))


def _sc_body(text_ref, ai0_ref, ai1_ref, ai2_ref, emb_ref, cat0_ref, cat1_ref,
             cat2_ref, sum_ref, cat_ref, idx_v, lidxa_v, lidxb_v, rowsa_v,
             rowsb_v, cat_idx_v, clidxa_v, clidxb_v, cat_rowsa_v, cat_rowsb_v,
             sum_v, catsum_v, gsem, csem):
    wid = lax.axis_index("s") * NC + lax.axis_index("c")
    base = wid * BPW

    pltpu.sync_copy(text_ref.at[pl.ds(base, BPW)], idx_v)
    ai_refs = (ai0_ref, ai1_ref, ai2_ref)
    for c in range(3):
        pltpu.sync_copy(ai_refs[c].at[pl.ds(base, BPW)], cat_idx_v.at[c])

    for c in range(3):
        for g in range(0, BPW, 16):
            la = _line16(cat_idx_v[c, pl.ds(g, 16)])
            clidxa_v[c, pl.ds(g, 16)] = la
            clidxb_v[c, pl.ds(g, 16)] = la + 1

    cat_tables = (cat0_ref, cat1_ref, cat2_ref)
    cat_descs = [
        pltpu.async_copy(cat_tables[c].at[clidxa_v.at[c]], cat_rowsa_v.at[c], csem)
        for c in range(3)
    ] + [
        pltpu.async_copy(cat_tables[c].at[clidxb_v.at[c]], cat_rowsb_v.at[c], csem)
        for c in range(3)
    ]

    def lidx_body(r, carry):
        for g in (0, 16, 32, L - 16):
            la = _line16(idx_v[r, pl.ds(g, 16)])
            lidxa_v[r, pl.ds(g, 16)] = la
            lidxb_v[r, pl.ds(g, 16)] = la + 1
        return carry

    lax.fori_loop(0, BPW, lidx_body, jnp.int32(0))

    def compute_row(r, chunk_base):
        a0 = [jnp.zeros((16,), jnp.float32) for _ in range(4)]
        a1 = [jnp.zeros((16,), jnp.float32) for _ in range(4)]
        for t in range(L):
            a0[t % 4] = a0[t % 4] + rowsa_v[r, t, :]
            a1[t % 4] = a1[t % 4] + rowsb_v[r, t, :]
        row = chunk_base + r
        sum_v[row, pl.ds(0, 16)] = (a0[0] + a0[1]) + (a0[2] + a0[3])
        sum_v[row, pl.ds(16, 16)] = (a1[0] + a1[1]) + (a1[2] + a1[3])

    for chunk in range(NCHUNK):
        cb = chunk * CHUNK
        descs = [
            pltpu.async_copy(emb_ref.at[lidxa_v.at[cb + r]], rowsa_v.at[r], gsem)
            for r in range(CHUNK)
        ] + [
            pltpu.async_copy(emb_ref.at[lidxb_v.at[cb + r]], rowsb_v.at[r], gsem)
            for r in range(CHUNK)
        ]
        for d in descs:
            d.wait()

        def body(r, carry):
            compute_row(r, cb)
            return carry

        lax.fori_loop(0, CHUNK, body, jnp.int32(0))

    for d in cat_descs:
        d.wait()

    def cat_body(r, carry):
        catsum_v[r, pl.ds(0, 16)] = (
            cat_rowsa_v[0, r, :] + cat_rowsa_v[1, r, :] + cat_rowsa_v[2, r, :]
        )
        catsum_v[r, pl.ds(16, 16)] = (
            cat_rowsb_v[0, r, :] + cat_rowsb_v[1, r, :] + cat_rowsb_v[2, r, :]
        )
        return carry

    lax.fori_loop(0, BPW, cat_body, jnp.int32(0))

    pltpu.sync_copy(sum_v, sum_ref.at[pl.ds(base, BPW)])
    pltpu.sync_copy(catsum_v, cat_ref.at[pl.ds(base, BPW)])


@jax.jit
def _sc_pool(encoded_text, ai0, ai1, ai2, emb_lines, cat_emb0, cat_emb1,
             cat_emb2):
    mesh = plsc.VectorSubcoreMesh(
        core_axis_name="c", subcore_axis_name="s", num_cores=NC, num_subcores=NS
    )
    return pl.kernel(
        _sc_body,
        out_type=(
            jax.ShapeDtypeStruct((B, D), jnp.float32),
            jax.ShapeDtypeStruct((B, D), jnp.float32),
        ),
        mesh=mesh,
        compiler_params=pltpu.CompilerParams(use_tc_tiling_on_sc=False),
        scratch_types=[
            pltpu.VMEM((BPW, L), jnp.int32),
            pltpu.VMEM((BPW, L), jnp.int32),
            pltpu.VMEM((BPW, L), jnp.int32),
            pltpu.VMEM((CHUNK, L, 16), jnp.float32),
            pltpu.VMEM((CHUNK, L, 16), jnp.float32),
            pltpu.VMEM((3, BPW), jnp.int32),
            pltpu.VMEM((3, BPW), jnp.int32),
            pltpu.VMEM((3, BPW), jnp.int32),
            pltpu.VMEM((3, BPW, 16), jnp.float32),
            pltpu.VMEM((3, BPW, 16), jnp.float32),
            pltpu.VMEM((BPW, D), jnp.float32),
            pltpu.VMEM((BPW, D), jnp.float32),
            pltpu.SemaphoreType.DMA,
            pltpu.SemaphoreType.DMA,
        ],
    )(encoded_text, ai0, ai1, ai2, emb_lines, cat_emb0, cat_emb1, cat_emb2)


VB = 65536
SUB = VB // 4


def _tr_body(x_ref, o_ref):
    x = x_ref[...]
    x4 = jnp.concatenate([x[:, j * SUB:(j + 1) * SUB] for j in range(4)],
                         axis=0)
    eye = (lax.broadcasted_iota(jnp.int32, (128, 128), 0)
           == lax.broadcasted_iota(jnp.int32, (128, 128), 1)).astype(jnp.float32)
    o_ref[...] = lax.dot_general(
        x4, eye, (((0,), (0,)), ((), ())),
        preferred_element_type=jnp.float32,
    )


@jax.jit
def _tc_transpose_pack(table_t):
    d, v = table_t.shape
    grid = (v + VB - 1) // VB
    return pl.pallas_call(
        _tr_body,
        grid=(grid,),
        in_specs=[pl.BlockSpec((d, VB), lambda i: (0, i))],
        out_specs=pl.BlockSpec((SUB, 128), lambda i: (i, 0)),
        out_shape=jax.ShapeDtypeStruct((grid * SUB, 128), jnp.float32),
    )(table_t)


def _head_body(text_ref, sum_ref, cat_ref, w_ref, b_ref, o_ref):
    cnt = jnp.sum((text_ref[...] != 0).astype(jnp.float32), axis=1,
                  keepdims=True)
    x = sum_ref[...] / cnt
    x = jnp.where(x != x, jnp.float32(0.0), x)
    x = jnp.minimum(jnp.maximum(x, -F32_MAX), F32_MAX)
    x = x + cat_ref[...]
    o_ref[...] = (
        jnp.dot(x, w_ref[...], preferred_element_type=jnp.float32) + b_ref[...]
    )


@jax.jit
def _tc_head(encoded_text, x_sum, cat_sum, w, b2d):
    bm = 512
    nc = w.shape[1]
    return pl.pallas_call(
        _head_body,
        grid=(B // bm,),
        in_specs=[
            pl.BlockSpec((bm, L), lambda i: (i, 0)),
            pl.BlockSpec((bm, D), lambda i: (i, 0)),
            pl.BlockSpec((bm, D), lambda i: (i, 0)),
            pl.BlockSpec((D, nc), lambda i: (0, 0)),
            pl.BlockSpec((1, nc), lambda i: (0, 0)),
        ],
        out_specs=pl.BlockSpec((bm, nc), lambda i: (i, 0)),
        out_shape=jax.ShapeDtypeStruct((B, nc), jnp.float32),
    )(encoded_text, x_sum, cat_sum, w, b2d)


def kernel(encoded_text, additional_inputs, emb_table, cat_emb0, cat_emb1,
           cat_emb2, fc_w, fc_b):
    text = encoded_text.astype(jnp.int32)
    ai = additional_inputs.astype(jnp.int32)
    emb16 = _tc_transpose_pack(emb_table.T).reshape(-1, 16)
    cat16 = [_tc_transpose_pack(t.T).reshape(-1, 16)
             for t in (cat_emb0, cat_emb1, cat_emb2)]
    x_sum, cat_sum = _sc_pool(text, ai[:, 0], ai[:, 1], ai[:, 2], emb16,
                              cat16[0], cat16[1], cat16[2])
    return _tc_head(text, x_sum, cat_sum, fc_w, fc_b.reshape(1, -1))


# speedup vs baseline: 1.9183x; 1.1152x over previous
import jax
import jax.numpy as jnp
from jax import lax
from jax.experimental import pallas as pl
from jax.experimental.pallas import tpu as pltpu
from jax.experimental.pallas import tpu_sc as plsc

B = 4096
L = 50
D = 32
NC = 2
NS = 16
NW = NC * NS
BPW = B // NW
CHUNK = 32
NCHUNK = BPW // CHUNK
F32_MAX = 3.4028235e38


def _line16(v):
    lvb = VB.bit_length() - 1
    lsub = SUB.bit_length() - 1
    return (((v >> lvb) << (lsub + 3)) + ((v & (SUB - 1)) << 3)
            + (((v >> lsub) & 3) << 1))


def _sc_text_body(text_ref, emb_ref, sum_ref, idx_v, lidxa_v, lidxb_v,
                  rowsa_v, rowsb_v, sum_v, gsem):
    wid = lax.axis_index("s") * NC + lax.axis_index("c")
    base = wid * BPW

    pltpu.sync_copy(text_ref.at[pl.ds(base, BPW)], idx_v)

    def lidx_body(r, carry):
        for g in (0, 16, 32, L - 16):
            la = _line16(idx_v[r, pl.ds(g, 16)])
            lidxa_v[r, pl.ds(g, 16)] = la
            lidxb_v[r, pl.ds(g, 16)] = la + 1
        return carry

    lax.fori_loop(0, BPW, lidx_body, jnp.int32(0))

    def compute_row(r, chunk_base):
        a0 = [jnp.zeros((16,), jnp.float32) for _ in range(4)]
        a1 = [jnp.zeros((16,), jnp.float32) for _ in range(4)]
        for t in range(L):
            a0[t % 4] = a0[t % 4] + rowsa_v[r, t, :]
            a1[t % 4] = a1[t % 4] + rowsb_v[r, t, :]
        row = chunk_base + r
        sum_v[row, pl.ds(0, 16)] = (a0[0] + a0[1]) + (a0[2] + a0[3])
        sum_v[row, pl.ds(16, 16)] = (a1[0] + a1[1]) + (a1[2] + a1[3])

    for chunk in range(NCHUNK):
        cb = chunk * CHUNK
        descs = [
            pltpu.async_copy(emb_ref.at[lidxa_v.at[cb + r]], rowsa_v.at[r], gsem)
            for r in range(CHUNK)
        ] + [
            pltpu.async_copy(emb_ref.at[lidxb_v.at[cb + r]], rowsb_v.at[r], gsem)
            for r in range(CHUNK)
        ]
        for d in descs:
            d.wait()

        def body(r, carry):
            compute_row(r, cb)
            return carry

        lax.fori_loop(0, CHUNK, body, jnp.int32(0))

    pltpu.sync_copy(sum_v, sum_ref.at[pl.ds(base, BPW)])


def _sc_cat_body(ai0_ref, ai1_ref, ai2_ref, cat0_ref, cat1_ref, cat2_ref,
                 cat_ref, cat_idx_v, clidxa_v, clidxb_v, cat_rowsa_v,
                 cat_rowsb_v, catsum_v, csem):
    wid = lax.axis_index("s") * NC + lax.axis_index("c")
    base = wid * BPW

    ai_refs = (ai0_ref, ai1_ref, ai2_ref)
    for c in range(3):
        pltpu.sync_copy(ai_refs[c].at[pl.ds(base, BPW)], cat_idx_v.at[c])

    for c in range(3):
        for g in range(0, BPW, 16):
            la = _line16(cat_idx_v[c, pl.ds(g, 16)])
            clidxa_v[c, pl.ds(g, 16)] = la
            clidxb_v[c, pl.ds(g, 16)] = la + 1

    cat_tables = (cat0_ref, cat1_ref, cat2_ref)
    cat_descs = [
        pltpu.async_copy(cat_tables[c].at[clidxa_v.at[c]], cat_rowsa_v.at[c], csem)
        for c in range(3)
    ] + [
        pltpu.async_copy(cat_tables[c].at[clidxb_v.at[c]], cat_rowsb_v.at[c], csem)
        for c in range(3)
    ]
    for d in cat_descs:
        d.wait()

    def cat_body(r, carry):
        catsum_v[r, pl.ds(0, 16)] = (
            cat_rowsa_v[0, r, :] + cat_rowsa_v[1, r, :] + cat_rowsa_v[2, r, :]
        )
        catsum_v[r, pl.ds(16, 16)] = (
            cat_rowsb_v[0, r, :] + cat_rowsb_v[1, r, :] + cat_rowsb_v[2, r, :]
        )
        return carry

    lax.fori_loop(0, BPW, cat_body, jnp.int32(0))

    pltpu.sync_copy(catsum_v, cat_ref.at[pl.ds(base, BPW)])


def _mesh():
    return plsc.VectorSubcoreMesh(
        core_axis_name="c", subcore_axis_name="s", num_cores=NC, num_subcores=NS
    )


@jax.jit
def _sc_text_pool(encoded_text, emb_lines):
    return pl.kernel(
        _sc_text_body,
        out_type=jax.ShapeDtypeStruct((B, D), jnp.float32),
        mesh=_mesh(),
        compiler_params=pltpu.CompilerParams(use_tc_tiling_on_sc=False),
        scratch_types=[
            pltpu.VMEM((BPW, L), jnp.int32),
            pltpu.VMEM((BPW, L), jnp.int32),
            pltpu.VMEM((BPW, L), jnp.int32),
            pltpu.VMEM((CHUNK, L, 16), jnp.float32),
            pltpu.VMEM((CHUNK, L, 16), jnp.float32),
            pltpu.VMEM((BPW, D), jnp.float32),
            pltpu.SemaphoreType.DMA,
        ],
    )(encoded_text, emb_lines)


@jax.jit
def _sc_cat_pool(ai0, ai1, ai2, cat0, cat1, cat2):
    return pl.kernel(
        _sc_cat_body,
        out_type=jax.ShapeDtypeStruct((B, D), jnp.float32),
        mesh=_mesh(),
        compiler_params=pltpu.CompilerParams(use_tc_tiling_on_sc=False),
        scratch_types=[
            pltpu.VMEM((3, BPW), jnp.int32),
            pltpu.VMEM((3, BPW), jnp.int32),
            pltpu.VMEM((3, BPW), jnp.int32),
            pltpu.VMEM((3, BPW, 16), jnp.float32),
            pltpu.VMEM((3, BPW, 16), jnp.float32),
            pltpu.VMEM((BPW, D), jnp.float32),
            pltpu.SemaphoreType.DMA,
        ],
    )(ai0, ai1, ai2, cat0, cat1, cat2)


VB = 65536
SUB = VB // 4


def _tr_body(x_ref, o_ref):
    x = x_ref[...]
    x4 = jnp.concatenate([x[:, j * SUB:(j + 1) * SUB] for j in range(4)],
                         axis=0)
    eye = (lax.broadcasted_iota(jnp.int32, (128, 128), 0)
           == lax.broadcasted_iota(jnp.int32, (128, 128), 1)).astype(jnp.float32)
    o_ref[...] = lax.dot_general(
        x4, eye, (((0,), (0,)), ((), ())),
        preferred_element_type=jnp.float32,
    )


@jax.jit
def _tc_transpose_pack(table_t):
    d, v = table_t.shape
    grid = (v + VB - 1) // VB
    return pl.pallas_call(
        _tr_body,
        grid=(grid,),
        in_specs=[pl.BlockSpec((d, VB), lambda i: (0, i))],
        out_specs=pl.BlockSpec((SUB, 128), lambda i: (i, 0)),
        out_shape=jax.ShapeDtypeStruct((grid * SUB, 128), jnp.float32),
    )(table_t)


def _head_body(text_ref, sum_ref, cat_ref, w_ref, b_ref, o_ref):
    cnt = jnp.sum((text_ref[...] != 0).astype(jnp.float32), axis=1,
                  keepdims=True)
    x = sum_ref[...] / cnt
    x = jnp.where(x != x, jnp.float32(0.0), x)
    x = jnp.minimum(jnp.maximum(x, -F32_MAX), F32_MAX)
    x = x + cat_ref[...]
    o_ref[...] = (
        jnp.dot(x, w_ref[...], preferred_element_type=jnp.float32) + b_ref[...]
    )


@jax.jit
def _tc_head(encoded_text, x_sum, cat_sum, w, b2d):
    bm = 512
    nc = w.shape[1]
    return pl.pallas_call(
        _head_body,
        grid=(B // bm,),
        in_specs=[
            pl.BlockSpec((bm, L), lambda i: (i, 0)),
            pl.BlockSpec((bm, D), lambda i: (i, 0)),
            pl.BlockSpec((bm, D), lambda i: (i, 0)),
            pl.BlockSpec((D, nc), lambda i: (0, 0)),
            pl.BlockSpec((1, nc), lambda i: (0, 0)),
        ],
        out_specs=pl.BlockSpec((bm, nc), lambda i: (i, 0)),
        out_shape=jax.ShapeDtypeStruct((B, nc), jnp.float32),
    )(encoded_text, x_sum, cat_sum, w, b2d)


def kernel(encoded_text, additional_inputs, emb_table, cat_emb0, cat_emb1,
           cat_emb2, fc_w, fc_b):
    text = encoded_text.astype(jnp.int32)
    ai = additional_inputs.astype(jnp.int32)
    emb16 = _tc_transpose_pack(emb_table.T).reshape(-1, 16)
    x_sum = _sc_text_pool(text, emb16)
    cat16 = [_tc_transpose_pack(t.T).reshape(-1, 16)
             for t in (cat_emb0, cat_emb1, cat_emb2)]
    cat_sum = _sc_cat_pool(ai[:, 0], ai[:, 1], ai[:, 2],
                           cat16[0], cat16[1], cat16[2])
    return _tc_head(text, x_sum, cat_sum, fc_w, fc_b.reshape(1, -1))


# speedup vs baseline: 1.9356x; 1.0090x over previous
import jax
import jax.numpy as jnp
from jax import lax
from jax.experimental import pallas as pl
from jax.experimental.pallas import tpu as pltpu
from jax.experimental.pallas import tpu_sc as plsc

B = 4096
L = 50
D = 32
NC = 2
NS = 16
NW = NC * NS
BPW = B // NW
CHUNK = 16
NCHUNK = BPW // CHUNK
F32_MAX = 3.4028235e38


def _line16(v):
    lvb = VB.bit_length() - 1
    lsub = SUB.bit_length() - 1
    return (((v >> lvb) << (lsub + 3)) + ((v & (SUB - 1)) << 3)
            + (((v >> lsub) & 3) << 1))


def _sc_text_body(text_ref, emb_ref, sum_ref, idx_v, lidxa_v, lidxb_v,
                  rowsa_v, rowsb_v, sum_v, gsem0, gsem1):
    wid = lax.axis_index("s") * NC + lax.axis_index("c")
    base = wid * BPW

    pltpu.sync_copy(text_ref.at[pl.ds(base, BPW)], idx_v)

    def lidx_body(r, carry):
        for g in (0, 16, 32, L - 16):
            la = _line16(idx_v[r, pl.ds(g, 16)])
            lidxa_v[r, pl.ds(g, 16)] = la
            lidxb_v[r, pl.ds(g, 16)] = la + 1
        return carry

    lax.fori_loop(0, BPW, lidx_body, jnp.int32(0))

    def compute_row(r, chunk_base, par):
        a0 = [jnp.zeros((16,), jnp.float32) for _ in range(4)]
        a1 = [jnp.zeros((16,), jnp.float32) for _ in range(4)]
        for t in range(L):
            a0[t % 4] = a0[t % 4] + rowsa_v[par, r, t, :]
            a1[t % 4] = a1[t % 4] + rowsb_v[par, r, t, :]
        row = chunk_base + r
        sum_v[row, pl.ds(0, 16)] = (a0[0] + a0[1]) + (a0[2] + a0[3])
        sum_v[row, pl.ds(16, 16)] = (a1[0] + a1[1]) + (a1[2] + a1[3])

    sems = (gsem0, gsem1)

    def fire(chunk):
        cb = chunk * CHUNK
        par = chunk % 2
        sem = sems[par]
        return [
            pltpu.async_copy(emb_ref.at[lidxa_v.at[cb + r]],
                             rowsa_v.at[par].at[r], sem)
            for r in range(CHUNK)
        ] + [
            pltpu.async_copy(emb_ref.at[lidxb_v.at[cb + r]],
                             rowsb_v.at[par].at[r], sem)
            for r in range(CHUNK)
        ]

    descs = fire(0)
    for chunk in range(NCHUNK):
        for d in descs:
            d.wait()
        if chunk + 1 < NCHUNK:
            descs = fire(chunk + 1)
        cb = chunk * CHUNK
        par = chunk % 2

        def body(r, carry):
            compute_row(r, cb, par)
            return carry

        lax.fori_loop(0, CHUNK, body, jnp.int32(0))

    pltpu.sync_copy(sum_v, sum_ref.at[pl.ds(base, BPW)])


def _sc_cat_body(ai0_ref, ai1_ref, ai2_ref, cat0_ref, cat1_ref, cat2_ref,
                 cat_ref, cat_idx_v, clidxa_v, clidxb_v, cat_rowsa_v,
                 cat_rowsb_v, catsum_v, csem):
    wid = lax.axis_index("s") * NC + lax.axis_index("c")
    base = wid * BPW

    ai_refs = (ai0_ref, ai1_ref, ai2_ref)
    for c in range(3):
        pltpu.sync_copy(ai_refs[c].at[pl.ds(base, BPW)], cat_idx_v.at[c])

    for c in range(3):
        for g in range(0, BPW, 16):
            la = _line16(cat_idx_v[c, pl.ds(g, 16)])
            clidxa_v[c, pl.ds(g, 16)] = la
            clidxb_v[c, pl.ds(g, 16)] = la + 1

    cat_tables = (cat0_ref, cat1_ref, cat2_ref)
    cat_descs = [
        pltpu.async_copy(cat_tables[c].at[clidxa_v.at[c]], cat_rowsa_v.at[c], csem)
        for c in range(3)
    ] + [
        pltpu.async_copy(cat_tables[c].at[clidxb_v.at[c]], cat_rowsb_v.at[c], csem)
        for c in range(3)
    ]
    for d in cat_descs:
        d.wait()

    def cat_body(r, carry):
        catsum_v[r, pl.ds(0, 16)] = (
            cat_rowsa_v[0, r, :] + cat_rowsa_v[1, r, :] + cat_rowsa_v[2, r, :]
        )
        catsum_v[r, pl.ds(16, 16)] = (
            cat_rowsb_v[0, r, :] + cat_rowsb_v[1, r, :] + cat_rowsb_v[2, r, :]
        )
        return carry

    lax.fori_loop(0, BPW, cat_body, jnp.int32(0))

    pltpu.sync_copy(catsum_v, cat_ref.at[pl.ds(base, BPW)])


def _mesh():
    return plsc.VectorSubcoreMesh(
        core_axis_name="c", subcore_axis_name="s", num_cores=NC, num_subcores=NS
    )


@jax.jit
def _sc_text_pool(encoded_text, emb_lines):
    return pl.kernel(
        _sc_text_body,
        out_type=jax.ShapeDtypeStruct((B, D), jnp.float32),
        mesh=_mesh(),
        compiler_params=pltpu.CompilerParams(use_tc_tiling_on_sc=False),
        scratch_types=[
            pltpu.VMEM((BPW, L), jnp.int32),
            pltpu.VMEM((BPW, L), jnp.int32),
            pltpu.VMEM((BPW, L), jnp.int32),
            pltpu.VMEM((2, CHUNK, L, 16), jnp.float32),
            pltpu.VMEM((2, CHUNK, L, 16), jnp.float32),
            pltpu.VMEM((BPW, D), jnp.float32),
            pltpu.SemaphoreType.DMA,
            pltpu.SemaphoreType.DMA,
        ],
    )(encoded_text, emb_lines)


@jax.jit
def _sc_cat_pool(ai0, ai1, ai2, cat0, cat1, cat2):
    return pl.kernel(
        _sc_cat_body,
        out_type=jax.ShapeDtypeStruct((B, D), jnp.float32),
        mesh=_mesh(),
        compiler_params=pltpu.CompilerParams(use_tc_tiling_on_sc=False),
        scratch_types=[
            pltpu.VMEM((3, BPW), jnp.int32),
            pltpu.VMEM((3, BPW), jnp.int32),
            pltpu.VMEM((3, BPW), jnp.int32),
            pltpu.VMEM((3, BPW, 16), jnp.float32),
            pltpu.VMEM((3, BPW, 16), jnp.float32),
            pltpu.VMEM((BPW, D), jnp.float32),
            pltpu.SemaphoreType.DMA,
        ],
    )(ai0, ai1, ai2, cat0, cat1, cat2)


VB = 65536
SUB = VB // 4


def _tr_body(x_ref, o_ref):
    x = x_ref[...]
    x4 = jnp.concatenate([x[:, j * SUB:(j + 1) * SUB] for j in range(4)],
                         axis=0)
    eye = (lax.broadcasted_iota(jnp.int32, (128, 128), 0)
           == lax.broadcasted_iota(jnp.int32, (128, 128), 1)).astype(jnp.float32)
    o_ref[...] = lax.dot_general(
        x4, eye, (((0,), (0,)), ((), ())),
        preferred_element_type=jnp.float32,
    )


@jax.jit
def _tc_transpose_pack(table_t):
    d, v = table_t.shape
    grid = (v + VB - 1) // VB
    return pl.pallas_call(
        _tr_body,
        grid=(grid,),
        in_specs=[pl.BlockSpec((d, VB), lambda i: (0, i))],
        out_specs=pl.BlockSpec((SUB, 128), lambda i: (i, 0)),
        out_shape=jax.ShapeDtypeStruct((grid * SUB, 128), jnp.float32),
    )(table_t)


def _head_body(text_ref, sum_ref, cat_ref, w_ref, b_ref, o_ref):
    cnt = jnp.sum((text_ref[...] != 0).astype(jnp.float32), axis=1,
                  keepdims=True)
    x = sum_ref[...] / cnt
    x = jnp.where(x != x, jnp.float32(0.0), x)
    x = jnp.minimum(jnp.maximum(x, -F32_MAX), F32_MAX)
    x = x + cat_ref[...]
    o_ref[...] = (
        jnp.dot(x, w_ref[...], preferred_element_type=jnp.float32) + b_ref[...]
    )


@jax.jit
def _tc_head(encoded_text, x_sum, cat_sum, w, b2d):
    bm = 512
    nc = w.shape[1]
    return pl.pallas_call(
        _head_body,
        grid=(B // bm,),
        in_specs=[
            pl.BlockSpec((bm, L), lambda i: (i, 0)),
            pl.BlockSpec((bm, D), lambda i: (i, 0)),
            pl.BlockSpec((bm, D), lambda i: (i, 0)),
            pl.BlockSpec((D, nc), lambda i: (0, 0)),
            pl.BlockSpec((1, nc), lambda i: (0, 0)),
        ],
        out_specs=pl.BlockSpec((bm, nc), lambda i: (i, 0)),
        out_shape=jax.ShapeDtypeStruct((B, nc), jnp.float32),
    )(encoded_text, x_sum, cat_sum, w, b2d)


def kernel(encoded_text, additional_inputs, emb_table, cat_emb0, cat_emb1,
           cat_emb2, fc_w, fc_b):
    text = encoded_text.astype(jnp.int32)
    ai = additional_inputs.astype(jnp.int32)
    emb16 = _tc_transpose_pack(emb_table.T).reshape(-1, 16)
    x_sum = _sc_text_pool(text, emb16)
    cat16 = [_tc_transpose_pack(t.T).reshape(-1, 16)
             for t in (cat_emb0, cat_emb1, cat_emb2)]
    cat_sum = _sc_cat_pool(ai[:, 0], ai[:, 1], ai[:, 2],
                           cat16[0], cat16[1], cat16[2])
    return _tc_head(text, x_sum, cat_sum, fc_w, fc_b.reshape(1, -1))
